# baseline (speedup 1.0000x reference)
import jax
import jax.numpy as jnp
from jax.experimental import pallas as pl
from jax.experimental.pallas import tpu as pltpu

H = 1024
F = 512
NR = 8
TOT = 12
EPAD = 128
T = 2048
BT = 512


def _moe_body(x_ref, wc_ref, bias_ref, gu_ref, dn_ref, o_ref):
    x = x_ref[...]
    logits = jax.lax.dot_general(
        x, wc_ref[...], (((1,), (1,)), ((), ())),
        preferred_element_type=jnp.float32)
    col = jax.lax.broadcasted_iota(jnp.int32, logits.shape, 1)
    valid = col < TOT
    logits = jnp.where(valid, logits, -1e30)
    m = jnp.max(logits, axis=1, keepdims=True)
    ex = jnp.where(valid, jnp.exp(logits - m), 0.0)
    scores = ex / jnp.sum(ex, axis=1, keepdims=True)
    choice = jnp.where(valid, scores + bias_ref[...], -1e30)
    m1 = jnp.max(choice, axis=1, keepdims=True)
    i1 = jnp.min(jnp.where(choice >= m1, col, TOT), axis=1, keepdims=True)
    choice2 = jnp.where(col == i1, -1e30, choice)
    m2 = jnp.max(choice2, axis=1, keepdims=True)
    i2 = jnp.min(jnp.where(choice2 >= m2, col, TOT), axis=1, keepdims=True)
    w = jnp.where((col == i1) | (col == i2), scores, 0.0)

    zw = jnp.sum(jnp.where(col >= NR, w, 0.0), axis=1, keepdims=True)
    acc = x * zw
    xb = x.astype(jnp.bfloat16)
    for e in range(NR):
        we = jnp.sum(jnp.where(col == e, w, 0.0), axis=1, keepdims=True)
        gu = jax.lax.dot_general(
            xb, gu_ref[e], (((1,), (1,)), ((), ())),
            preferred_element_type=jnp.float32)
        g = gu[:, :F]
        u = gu[:, F:]
        h = (g * jax.lax.logistic(g)) * u * we
        y = jax.lax.dot_general(
            h.astype(jnp.bfloat16), dn_ref[e], (((1,), (1,)), ((), ())),
            preferred_element_type=jnp.float32)
        acc = acc + y
    o_ref[...] = acc


def kernel(hidden_states, classifier_weight, e_score_correction_bias,
           gate_up_proj, down_proj):
    x = hidden_states.reshape(-1, H)
    wc = jnp.zeros((EPAD, H), jnp.float32).at[:TOT].set(classifier_weight)
    bias = jnp.zeros((1, EPAD), jnp.float32).at[0, :TOT].set(
        e_score_correction_bias)

    out = pl.pallas_call(
        _moe_body,
        grid=(T // BT,),
        in_specs=[
            pl.BlockSpec((BT, H), lambda i: (i, 0)),
            pl.BlockSpec((EPAD, H), lambda i: (0, 0)),
            pl.BlockSpec((1, EPAD), lambda i: (0, 0)),
            pl.BlockSpec((NR, 2 * F, H), lambda i: (0, 0, 0)),
            pl.BlockSpec((NR, H, F), lambda i: (0, 0, 0)),
        ],
        out_specs=pl.BlockSpec((BT, H), lambda i: (i, 0)),
        out_shape=jax.ShapeDtypeStruct((T, H), jnp.float32),
        compiler_params=pltpu.CompilerParams(
            dimension_semantics=("arbitrary",)),
    )(x, wc, bias, gate_up_proj[:NR].astype(jnp.bfloat16),
      down_proj.astype(jnp.bfloat16))
    return out.reshape(hidden_states.shape)


# speedup vs baseline: 1.0530x; 1.0530x over previous
import jax
import jax.numpy as jnp
from jax.experimental import pallas as pl
from jax.experimental.pallas import tpu as pltpu

H = 1024
F = 512
NR = 8
TOT = 12
EPAD = 128
T = 2048
BT = 512


def _moe_body(x_ref, wc_ref, bias_ref, gu_ref, dn_ref, o_ref):
    x = x_ref[...]
    logits = jax.lax.dot_general(
        x, wc_ref[...], (((1,), (1,)), ((), ())),
        preferred_element_type=jnp.float32)
    col = jax.lax.broadcasted_iota(jnp.int32, logits.shape, 1)
    valid = col < TOT
    logits = jnp.where(valid, logits, -1e30)
    m = jnp.max(logits, axis=1, keepdims=True)
    ex = jnp.where(valid, jnp.exp(logits - m), 0.0)
    scores = ex / jnp.sum(ex, axis=1, keepdims=True)
    choice = jnp.where(valid, scores + bias_ref[...], -1e30)
    m1 = jnp.max(choice, axis=1, keepdims=True)
    i1 = jnp.min(jnp.where(choice >= m1, col, TOT), axis=1, keepdims=True)
    choice2 = jnp.where(col == i1, -1e30, choice)
    m2 = jnp.max(choice2, axis=1, keepdims=True)
    i2 = jnp.min(jnp.where(choice2 >= m2, col, TOT), axis=1, keepdims=True)
    w = jnp.where((col == i1) | (col == i2), scores, 0.0)

    zw = jnp.sum(jnp.where(col >= NR, w, 0.0), axis=1, keepdims=True)
    acc = x * zw
    xb = x.astype(jnp.bfloat16)
    for e in range(NR):
        we = jnp.sum(jnp.where(col == e, w, 0.0), axis=1, keepdims=True)
        gu = jax.lax.dot_general(
            xb, gu_ref[e], (((1,), (1,)), ((), ())),
            preferred_element_type=jnp.float32)
        g = gu[:, :F]
        u = gu[:, F:]
        h = (g * jax.lax.logistic(g)) * u
        y = jax.lax.dot_general(
            h.astype(jnp.bfloat16), dn_ref[e], (((1,), (1,)), ((), ())),
            preferred_element_type=jnp.float32)
        acc = acc + y * we
    o_ref[...] = acc


def kernel(hidden_states, classifier_weight, e_score_correction_bias,
           gate_up_proj, down_proj):
    x = hidden_states.reshape(-1, H)
    wc = jnp.zeros((EPAD, H), jnp.float32).at[:TOT].set(classifier_weight)
    bias = jnp.zeros((1, EPAD), jnp.float32).at[0, :TOT].set(
        e_score_correction_bias)

    out = pl.pallas_call(
        _moe_body,
        grid=(T // BT,),
        in_specs=[
            pl.BlockSpec((BT, H), lambda i: (i, 0)),
            pl.BlockSpec((EPAD, H), lambda i: (0, 0)),
            pl.BlockSpec((1, EPAD), lambda i: (0, 0)),
            pl.BlockSpec((NR, 2 * F, H), lambda i: (0, 0, 0)),
            pl.BlockSpec((NR, H, F), lambda i: (0, 0, 0)),
        ],
        out_specs=pl.BlockSpec((BT, H), lambda i: (i, 0)),
        out_shape=jax.ShapeDtypeStruct((T, H), jnp.float32),
        compiler_params=pltpu.CompilerParams(
            dimension_semantics=("arbitrary",)),
    )(x, wc, bias, gate_up_proj[:NR].astype(jnp.bfloat16),
      down_proj.astype(jnp.bfloat16))
    return out.reshape(hidden_states.shape)
